# baseline (speedup 1.0000x reference)
import functools

import jax
import jax.numpy as jnp
from jax.experimental import pallas as pl
from jax.experimental.pallas import tpu as pltpu

EPS = 1e-5


def _round_up(v, m):
    return (v + m - 1) // m * m


def _bn(y, gamma, beta, n_rows, *, relu):
    inv_n = 1.0 / n_rows
    mean = jnp.sum(y, axis=0, keepdims=True) * inv_n
    var = jnp.sum(y * y, axis=0, keepdims=True) * inv_n - mean * mean
    var = jnp.maximum(var, 0.0)
    scale = jax.lax.rsqrt(var + EPS) * gamma
    out = (y - mean) * scale + beta
    return jnp.maximum(out, 0.0) if relu else out


def _conv1_kernel(xb_ref, w1_ref, g1_ref, b1_ref, z1_ref, *, M):
    y = jnp.dot(xb_ref[...], w1_ref[...], preferred_element_type=jnp.float32)
    z = _bn(y, g1_ref[...], b1_ref[...], M, relu=True)
    z1_ref[...] = z.astype(jnp.bfloat16)


def _conv2_kernel(z1_ref, w2_ref, g1_ref, b1_ref, z2_ref, xp_ref,
                  *, N, H, W, pad_off):
    M = N * H * W
    Mpad, Cmid_p = xp_ref.shape

    xp_ref[0:pad_off, :] = jnp.zeros((pad_off, Cmid_p), xp_ref.dtype)
    xp_ref[pad_off + M:Mpad, :] = jnp.zeros((Mpad - pad_off - M, Cmid_p),
                                            xp_ref.dtype)
    xp_ref[pad_off:pad_off + M, :] = z1_ref[...]

    ii = jax.lax.broadcasted_iota(jnp.int32, (M, 1), 0)
    yy = (ii % (H * W)) // W
    xx = ii % W
    row_ok = {-1: yy >= 1, 1: yy < H - 1}
    col_ok = {-1: xx >= 1, 1: xx < W - 1}

    acc = jnp.dot(xp_ref[pad_off:pad_off + M, :], w2_ref[1, 1, :, :],
                  preferred_element_type=jnp.float32)
    for dy in (-1, 0, 1):
        for dx in (-1, 0, 1):
            if dy == 0 and dx == 0:
                continue
            start = pad_off + dy * W + dx
            tap = jnp.dot(xp_ref[start:start + M, :], w2_ref[dy + 1, dx + 1, :, :],
                          preferred_element_type=jnp.float32)
            if dy == 0:
                ok = col_ok[dx]
            elif dx == 0:
                ok = row_ok[dy]
            else:
                ok = jnp.logical_and(row_ok[dy], col_ok[dx])
            acc = acc + jnp.where(ok, tap, 0.0)

    z = _bn(acc, g1_ref[...], b1_ref[...], M, relu=True)
    z2_ref[...] = z.astype(jnp.bfloat16)


def _conv3_kernel(z2_ref, w3_ref, g2_ref, b2_ref, x_ref, o_ref, *, M):
    y = jnp.dot(z2_ref[...], w3_ref[...], preferred_element_type=jnp.float32)
    y = _bn(y, g2_ref[...], b2_ref[...], M, relu=False)
    o_ref[...] = jnp.maximum(y + x_ref[...], 0.0)


@jax.jit
def _forward(x_nchw, w1, w2, w3, g1, b1, g2, b2):
    N, Cin, H, W = x_nchw.shape
    Cin_p, Cmid_p = w1.shape
    M = N * H * W
    pad_off = _round_up(W + 1, 8)
    Mpad = _round_up(pad_off + M + W + 1, 8)

    x_flat = jnp.transpose(x_nchw, (0, 2, 3, 1)).astype(jnp.float32).reshape(M, Cin)
    if Cin_p != Cin:
        x_flat = jnp.zeros((M, Cin_p), jnp.float32).at[:, :Cin].set(x_flat)
    xb = x_flat.astype(jnp.bfloat16)

    cp = pltpu.CompilerParams(dimension_semantics=("parallel",),
                              vmem_limit_bytes=64 << 20)

    GA = 2
    CA = Cmid_p // GA
    z1 = pl.pallas_call(
        functools.partial(_conv1_kernel, M=M),
        out_shape=jax.ShapeDtypeStruct((M, Cmid_p), jnp.bfloat16),
        grid=(GA,),
        in_specs=[
            pl.BlockSpec((M, Cin_p), lambda i: (0, 0)),
            pl.BlockSpec((Cin_p, CA), lambda i: (0, i)),
            pl.BlockSpec((1, CA), lambda i: (0, i)),
            pl.BlockSpec((1, CA), lambda i: (0, i)),
        ],
        out_specs=pl.BlockSpec((M, CA), lambda i: (0, i)),
        compiler_params=cp,
    )(xb, w1, g1, b1)

    GB = 2
    CB = Cmid_p // GB
    z2 = pl.pallas_call(
        functools.partial(_conv2_kernel, N=N, H=H, W=W, pad_off=pad_off),
        out_shape=jax.ShapeDtypeStruct((M, Cmid_p), jnp.bfloat16),
        grid=(GB,),
        in_specs=[
            pl.BlockSpec((M, Cmid_p), lambda i: (0, 0)),
            pl.BlockSpec((3, 3, Cmid_p, CB), lambda i: (0, 0, 0, i)),
            pl.BlockSpec((1, CB), lambda i: (0, i)),
            pl.BlockSpec((1, CB), lambda i: (0, i)),
        ],
        out_specs=pl.BlockSpec((M, CB), lambda i: (0, i)),
        scratch_shapes=[pltpu.VMEM((Mpad, Cmid_p), jnp.bfloat16)],
        compiler_params=cp,
    )(z1, w2, g1, b1)

    GC = 2
    CC = Cin_p // GC
    out = pl.pallas_call(
        functools.partial(_conv3_kernel, M=M),
        out_shape=jax.ShapeDtypeStruct((M, Cin_p), jnp.float32),
        grid=(GC,),
        in_specs=[
            pl.BlockSpec((M, Cmid_p), lambda i: (0, 0)),
            pl.BlockSpec((Cmid_p, CC), lambda i: (0, i)),
            pl.BlockSpec((1, CC), lambda i: (0, i)),
            pl.BlockSpec((1, CC), lambda i: (0, i)),
            pl.BlockSpec((M, CC), lambda i: (0, i)),
        ],
        out_specs=pl.BlockSpec((M, CC), lambda i: (0, i)),
        compiler_params=cp,
    )(z2, w3, g2, b2, x_flat)

    y = out[:, :Cin].reshape(N, H, W, Cin)
    return jnp.transpose(y, (0, 3, 1, 2))


def kernel(x, w1, w2, w3, g1, b1, g2, b2):
    return _forward(x, w1, w2, w3, g1, b1, g2, b2)
